# baseline (speedup 1.0000x reference)
import functools

import jax
import jax.numpy as jnp
from jax import lax
from jax.experimental import pallas as pl
from jax.experimental.pallas import tpu as pltpu
from jax.experimental.pallas import tpu_sc as plsc

_BLOCK = 2560


def _node_scalar_body(x_ref, we1_ref, be1_ref, we2_ref, be2_ref,
                      wa1_ref, ba1_ref, wa2_ref, ba2_ref, o_ref):
    x = x_ref[...]
    h = jnp.maximum(
        jnp.dot(x, we1_ref[...], preferred_element_type=jnp.float32)
        + be1_ref[...], 0.0)
    e = (jnp.dot(h, we2_ref[...], preferred_element_type=jnp.float32)
         + be2_ref[...])
    a = jnp.maximum(
        jnp.dot(e, wa1_ref[...], preferred_element_type=jnp.float32)
        + ba1_ref[...], 0.0)
    sblk = (jnp.dot(a, wa2_ref[...], preferred_element_type=jnp.float32)
            + ba2_ref[0, 0])
    i = pl.program_id(0)
    o_ref[pl.ds(i * _BLOCK, _BLOCK)] = sblk[:, 0]


def _node_scalars(nfeat, we1, be1, we2, be2, wa1, ba1, wa2, ba2):
    n, d = nfeat.shape

    def full(arr):
        return pl.BlockSpec(arr.shape, lambda i: (0,) * arr.ndim)

    grid = pl.cdiv(n, _BLOCK)
    n_out = grid * _BLOCK
    return pl.pallas_call(
        _node_scalar_body,
        grid=(grid,),
        in_specs=[
            pl.BlockSpec((_BLOCK, d), lambda i: (i, 0)),
            full(we1), full(be1), full(we2), full(be2),
            full(wa1), full(ba1), full(wa2), full(ba2),
        ],
        out_specs=pl.BlockSpec((n_out,), lambda i: (0,)),
        out_shape=jax.ShapeDtypeStruct((n_out,), jnp.float32),
    )(nfeat, we1, be1, we2, be2, wa1, ba1, wa2, ba2)


def _edge_means(s_vec, idx4t, idx8t):
    info = plsc.get_sparse_core_info()
    nc, ns, l = info.num_cores, info.num_subcores, info.num_lanes
    nw = nc * ns
    n_pad = s_vec.shape[0]
    s4, e4 = idx4t.shape
    s8, e8 = idx8t.shape
    ch4, ch8 = e4 // nw, e8 // nw
    mesh = plsc.VectorSubcoreMesh(core_axis_name="c", subcore_axis_name="s")

    @functools.partial(
        pl.kernel,
        mesh=mesh,
        out_type=jax.ShapeDtypeStruct((e4 + e8,), jnp.float32),
        compiler_params=pltpu.CompilerParams(needs_layout_passes=False),
        scratch_types=[
            pltpu.VMEM((n_pad,), jnp.float32),
            pltpu.VMEM((s4, ch4), jnp.int32),
            pltpu.VMEM((s8, ch8), jnp.int32),
            pltpu.VMEM((ch4,), jnp.float32),
            pltpu.VMEM((ch8,), jnp.float32),
            pltpu.SemaphoreType.DMA,
        ],
    )
    def k(s_hbm, i4_hbm, i8_hbm, out_hbm, s_v, i4_v, i8_v, o4_v, o8_v, sem):
        wid = lax.axis_index("s") * nc + lax.axis_index("c")
        c1 = pltpu.async_copy(s_hbm, s_v, sem)
        c2 = pltpu.async_copy(i4_hbm.at[:, pl.ds(wid * ch4, ch4)], i4_v, sem)
        c3 = pltpu.async_copy(i8_hbm.at[:, pl.ds(wid * ch8, ch8)], i8_v, sem)
        c1.wait()
        c2.wait()
        c3.wait()
        for t in range(ch4 // l):
            acc = jnp.zeros((l,), jnp.float32)
            for j in range(s4):
                acc = acc + plsc.load_gather(s_v, [i4_v[j, pl.ds(t * l, l)]])
            o4_v[pl.ds(t * l, l)] = acc * (1.0 / s4)
        for t in range(ch8 // l):
            acc = jnp.zeros((l,), jnp.float32)
            for j in range(s8):
                acc = acc + plsc.load_gather(s_v, [i8_v[j, pl.ds(t * l, l)]])
            o8_v[pl.ds(t * l, l)] = acc * (1.0 / s8)
        pltpu.sync_copy(o4_v, out_hbm.at[pl.ds(wid * ch4, ch4)])
        pltpu.sync_copy(o8_v, out_hbm.at[pl.ds(e4 + wid * ch8, ch8)])

    return k(s_vec, idx4t, idx8t)


def kernel(nfeat, hedges_s4, hedges_s8, W_e1, b_e1, W_e2, b_e2,
           W_a1, b_a1, W_a2, b_a2):
    s = _node_scalars(
        nfeat, W_e1, b_e1.reshape(1, -1), W_e2, b_e2.reshape(1, -1),
        W_a1, b_a1.reshape(1, -1), W_a2, b_a2.reshape(1, 1))
    idx4t = jnp.asarray(hedges_s4.T, jnp.int32)
    idx8t = jnp.asarray(hedges_s8.T, jnp.int32)
    return _edge_means(s, idx4t, idx8t)
